# baseline (speedup 1.0000x reference)
import functools

import jax
import jax.numpy as jnp
from jax import lax
from jax.experimental import pallas as pl
from jax.experimental.pallas import tpu as pltpu
from jax.experimental.pallas import tpu_sc as plsc

NUM_CORES = 2
NUM_SUBCORES = 16
NUM_WORKERS = NUM_CORES * NUM_SUBCORES
LANES = 16
CHUNK = 1600
NBUF = 2
PACK = 4


def _embed_sc(xf, tab_flat, n, dim):
    mesh = plsc.VectorSubcoreMesh(core_axis_name="c", subcore_axis_name="s")
    vd = tab_flat.shape[0]
    n_per_worker = n // NUM_WORKERS
    n_chunks = n_per_worker // CHUNK
    n_blocks = n_chunks // NBUF
    groups = CHUNK // LANES
    cp = CHUNK // PACK
    pd = PACK * dim

    @functools.partial(
        pl.kernel,
        out_type=jax.ShapeDtypeStruct((n // PACK, pd), jnp.float32),
        mesh=mesh,
        scratch_types=[
            pltpu.VMEM((vd,), jnp.float32),
            pltpu.VMEM((CHUNK,), jnp.int32),
            pltpu.VMEM((CHUNK,), jnp.int32),
            pltpu.VMEM((cp, pd), jnp.float32),
            pltpu.VMEM((cp, pd), jnp.float32),
            pltpu.SemaphoreType.DMA,
            pltpu.SemaphoreType.DMA,
            pltpu.SemaphoreType.DMA,
            pltpu.SemaphoreType.DMA,
        ],
        compiler_params=pltpu.CompilerParams(needs_layout_passes=False),
    )
    def k(xf_hbm, tab_hbm, out_hbm, tab_v, idx0, idx1, out0, out1, si0, si1, so0, so1):
        idx_b = (idx0, idx1)
        out_b = (out0, out1)
        sem_i = (si0, si1)
        sem_o = (so0, so1)
        wid = lax.axis_index("s") * NUM_CORES + lax.axis_index("c")
        base = wid * n_per_worker
        pbase = wid * (n_per_worker // PACK)
        pltpu.sync_copy(tab_hbm, tab_v)

        for b in range(NBUF):
            pltpu.async_copy(
                xf_hbm.at[pl.ds(base + b * CHUNK, CHUNK)], idx_b[b], sem_i[b]
            )

        def blk_body(blk, carry):
            for b in range(NBUF):
                i = blk * NBUF + b
                poff = pbase + i * cp
                pltpu.make_async_copy(
                    xf_hbm.at[pl.ds(base + i * CHUNK, CHUNK)], idx_b[b], sem_i[b]
                ).wait()

                @pl.when(blk > 0)
                def _wait_out():
                    pltpu.make_async_copy(
                        out_b[b], out_hbm.at[pl.ds(0, cp), :], sem_o[b]
                    ).wait()

                def grp_body(g, pos4):
                    toffv = idx_b[b][pl.ds(g * LANES, LANES)] * dim
                    for j in range(LANES):
                        toff = toffv[j]
                        r4 = pos4 + (j // PACK)
                        lane = (j % PACK) * dim
                        for h in range(dim // LANES):
                            out_b[b][r4, pl.ds(lane + h * LANES, LANES)] = tab_v[
                                pl.ds(toff + h * LANES, LANES)
                            ]
                    return pos4 + (LANES // PACK)

                lax.fori_loop(0, groups, grp_body, 0)
                pltpu.async_copy(
                    out_b[b], out_hbm.at[pl.ds(poff, cp), :], sem_o[b]
                )

                @pl.when(blk < n_blocks - 1)
                def _prefetch():
                    pltpu.async_copy(
                        xf_hbm.at[pl.ds(base + (i + NBUF) * CHUNK, CHUNK)],
                        idx_b[b],
                        sem_i[b],
                    )

            return carry

        lax.fori_loop(0, n_blocks, blk_body, 0)
        for b in range(NBUF):
            pltpu.make_async_copy(
                out_b[b], out_hbm.at[pl.ds(0, cp), :], sem_o[b]
            ).wait()

    return k(xf, tab_flat)


def kernel(x, table):
    batch, seq = x.shape
    vocab, dim = table.shape
    n = batch * seq
    assert n % (NUM_WORKERS * CHUNK * NBUF) == 0
    xf = x.reshape(n).astype(jnp.int32)
    out = _embed_sc(xf, table.reshape(vocab * dim), n, dim)
    return out.reshape(batch, seq, dim)


# speedup vs baseline: 2.1102x; 2.1102x over previous
import functools

import jax
import jax.numpy as jnp
from jax import lax
from jax.experimental import pallas as pl
from jax.experimental.pallas import tpu as pltpu
from jax.experimental.pallas import tpu_sc as plsc

NUM_CORES = 2
NUM_SUBCORES = 16
NUM_WORKERS = NUM_CORES * NUM_SUBCORES
LANES = 16
CHUNK = 320
NBUF = 2
JB = 8


def _embed_sc(xf, tab_flat, n, dim):
    mesh = plsc.VectorSubcoreMesh(core_axis_name="c", subcore_axis_name="s")
    vd = tab_flat.shape[0]
    n_per_worker = n // NUM_WORKERS
    n_chunks = n_per_worker // CHUNK
    n_blocks = n_chunks // NBUF
    groups = CHUNK // LANES

    @functools.partial(
        pl.kernel,
        out_type=jax.ShapeDtypeStruct((n, dim), jnp.float32),
        mesh=mesh,
        scratch_types=[
            pltpu.VMEM((vd,), jnp.float32),
            pltpu.VMEM((CHUNK,), jnp.int32),
            pltpu.VMEM((CHUNK,), jnp.int32),
            pltpu.VMEM((CHUNK, dim), jnp.float32),
            pltpu.VMEM((CHUNK, dim), jnp.float32),
            pltpu.SemaphoreType.DMA,
            pltpu.SemaphoreType.DMA,
            pltpu.SemaphoreType.DMA,
            pltpu.SemaphoreType.DMA,
        ],
        compiler_params=pltpu.CompilerParams(needs_layout_passes=False),
    )
    def k(xf_hbm, tab_hbm, out_hbm, tab_v, idx0, idx1, out0, out1, si0, si1, so0, so1):
        idx_b = (idx0, idx1)
        out_b = (out0, out1)
        sem_i = (si0, si1)
        sem_o = (so0, so1)
        wid = lax.axis_index("s") * NUM_CORES + lax.axis_index("c")
        base = wid * n_per_worker
        pltpu.sync_copy(tab_hbm, tab_v)

        for b in range(NBUF):
            pltpu.async_copy(
                xf_hbm.at[pl.ds(base + b * CHUNK, CHUNK)], idx_b[b], sem_i[b]
            )

        def blk_body(blk, carry):
            for b in range(NBUF):
                i = blk * NBUF + b
                off = base + i * CHUNK
                pltpu.make_async_copy(
                    xf_hbm.at[pl.ds(off, CHUNK)], idx_b[b], sem_i[b]
                ).wait()

                @pl.when(blk > 0)
                def _wait_out():
                    pltpu.make_async_copy(
                        out_b[b], out_hbm.at[pl.ds(0, CHUNK), :], sem_o[b]
                    ).wait()

                def grp_body(g, pos):
                    toffv = idx_b[b][pl.ds(g * LANES, LANES)] * dim
                    for jb in range(0, LANES, JB):
                        vals = []
                        for j in range(jb, jb + JB):
                            toff = toffv[j]
                            vals.append(
                                [
                                    tab_v[pl.ds(toff + h * LANES, LANES)]
                                    for h in range(dim // LANES)
                                ]
                            )
                        for jj, j in enumerate(range(jb, jb + JB)):
                            r = pos + j
                            for h in range(dim // LANES):
                                out_b[b][r, pl.ds(h * LANES, LANES)] = vals[jj][h]
                    return pos + LANES

                lax.fori_loop(0, groups, grp_body, 0)
                pltpu.async_copy(
                    out_b[b], out_hbm.at[pl.ds(off, CHUNK), :], sem_o[b]
                )

                @pl.when(blk < n_blocks - 1)
                def _prefetch():
                    pltpu.async_copy(
                        xf_hbm.at[pl.ds(off + NBUF * CHUNK, CHUNK)],
                        idx_b[b],
                        sem_i[b],
                    )

            return carry

        lax.fori_loop(0, n_blocks, blk_body, 0)
        for b in range(NBUF):
            pltpu.make_async_copy(
                out_b[b], out_hbm.at[pl.ds(0, CHUNK), :], sem_o[b]
            ).wait()

    return k(xf, tab_flat)


def kernel(x, table):
    batch, seq = x.shape
    vocab, dim = table.shape
    n = batch * seq
    assert n % (NUM_WORKERS * CHUNK * NBUF) == 0
    xf = x.reshape(n).astype(jnp.int32)
    out = _embed_sc(xf, table.reshape(vocab * dim), n, dim)
    return out.reshape(batch, seq, dim)
